# baseline (speedup 1.0000x reference)
import jax
import jax.numpy as jnp
from jax.experimental import pallas as pl
from jax.experimental.pallas import tpu as pltpu

DM = 64
CB = 1024
BN_EPS = 1e-5
NPIX = 4 * 224 * 224
NTOK = 4 * 112 * 112


def _conv1_h(xa_ref, xb_ref, w1_ref):
    win = jnp.concatenate([xa_ref[0], xb_ref[0, :2]], axis=0)
    x27 = jnp.concatenate([win[ky:ky + 16] for ky in range(3)],
                          axis=2).reshape(16 * 224, 27)
    return jnp.dot(x27, w1_ref[...], preferred_element_type=jnp.float32)


def _stats1_kernel(xa_ref, xb_ref, w1_ref, s1_ref, s2_ref):
    i = pl.program_id(0)

    @pl.when(i == 0)
    def _():
        s1_ref[...] = jnp.zeros_like(s1_ref)
        s2_ref[...] = jnp.zeros_like(s2_ref)

    h = _conv1_h(xa_ref, xb_ref, w1_ref)
    s1_ref[...] += jnp.sum(h, axis=0, keepdims=True)
    s2_ref[...] += jnp.sum(h * h, axis=0, keepdims=True)


def _encode_kernel(xa_ref, xb_ref, w1_ref, g1_ref, b1_ref, s1_ref, s2_ref,
                   cbt_ref, cb_ref, idx_ref, quant_ref, loss_ref):
    i = pl.program_id(0)
    mean = s1_ref[...] / NPIX
    var = s2_ref[...] / NPIX - mean * mean
    scale = g1_ref[...] * jax.lax.rsqrt(var + BN_EPS)
    shift = b1_ref[...] - mean * scale

    h = _conv1_h(xa_ref, xb_ref, w1_ref)
    y = jnp.maximum(h * scale + shift, 0.0)
    y3 = y.reshape(16, 224, DM)
    p = jnp.max(y3.reshape(16, 112, 2, DM), axis=2)
    p = jnp.max(p.reshape(8, 2, 112, DM), axis=1)
    tok = p.reshape(896, DM)

    tsq = jnp.sum(tok * tok, axis=1, keepdims=True)
    csq = jnp.sum(cbt_ref[...] * cbt_ref[...], axis=0, keepdims=True)
    dist = tsq - jnp.dot(tok, cbt_ref[...] * 2.0,
                         preferred_element_type=jnp.float32) + csq
    mind = jnp.min(dist, axis=1, keepdims=True)
    lanes = jax.lax.broadcasted_iota(jnp.int32, dist.shape, 1)
    idx = jnp.min(jnp.where(dist == mind, lanes, jnp.int32(1 << 30)), axis=1)
    idx_ref[...] = idx.reshape(1, 1, 896)

    onehot = (lanes == idx[:, None]).astype(jnp.float32)
    quant_ref[...] = jnp.dot(onehot, cb_ref[...],
                             preferred_element_type=jnp.float32
                             ).reshape(1, 896, DM)

    @pl.when(i == 0)
    def _():
        loss_ref[...] = jnp.zeros_like(loss_ref)

    loss_ref[...] += jnp.sum(mind)


def _decode_kernel(q_ref, qprev_ref, qnext_ref, w2_ref,
                   p00_ref, p01_ref, p10_ref, p11_ref, s1_ref, s2_ref):
    i = pl.program_id(0)
    rc = i % 14
    cur = q_ref[0]
    top = jnp.where(rc > 0, qprev_ref[0, 7], 0.0)
    bot = jnp.where(rc < 13, qnext_ref[0, 0], 0.0)
    qwin = jnp.concatenate([top[None], cur, bot[None]], axis=0)
    zc = jnp.zeros((10, 1, DM), jnp.float32)
    qs = jnp.concatenate([zc, qwin, zc], axis=1)

    @pl.when(i == 0)
    def _():
        s1_ref[...] = jnp.zeros_like(s1_ref)
        s2_ref[...] = jnp.zeros_like(s2_ref)

    outs = [p00_ref, p01_ref, p10_ref, p11_ref]
    for py in range(2):
        for px in range(2):
            acc = jnp.zeros((8 * 112, 4), jnp.float32)
            for ia in range(2):
                for ja in range(2):
                    sl = jax.lax.slice(
                        qs, (py + ia, px + ja, 0),
                        (py + ia + 8, px + ja + 112, DM)).reshape(8 * 112, DM)
                    acc = acc + jnp.dot(sl, w2_ref[py, px, ia, ja],
                                        precision=jax.lax.Precision.HIGHEST,
                                        preferred_element_type=jnp.float32)
            outs[2 * py + px][...] = acc.reshape(1, 8, 112, 4)
            s1_ref[...] += jnp.sum(acc, axis=0, keepdims=True)
            s2_ref[...] += jnp.sum(acc * acc, axis=0, keepdims=True)


def _finish_kernel(r00_ref, r01_ref, r10_ref, r11_ref, g2_ref, b2_ref,
                   s1_ref, s2_ref, f00_ref, f01_ref, f10_ref, f11_ref):
    mean = s1_ref[...] / NPIX
    var = s2_ref[...] / NPIX - mean * mean
    scale = g2_ref[...] * jax.lax.rsqrt(var + BN_EPS)
    shift = b2_ref[...] - mean * scale
    f00_ref[...] = jnp.tanh(r00_ref[...] * scale + shift)
    f01_ref[...] = jnp.tanh(r01_ref[...] * scale + shift)
    f10_ref[...] = jnp.tanh(r10_ref[...] * scale + shift)
    f11_ref[...] = jnp.tanh(r11_ref[...] * scale + shift)


def _full(shape):
    n = len(shape)
    return pl.BlockSpec(shape, lambda i: (0,) * n)


def kernel(x, conv1_w, conv1_b, bn1_g, bn1_b, codebook,
           conv2_w, conv2_b, bn2_g, bn2_b):
    f32 = jnp.float32
    xt = jnp.transpose(x, (0, 2, 3, 1))
    xw = jnp.pad(xt, ((0, 0), (0, 0), (1, 1), (0, 0)))
    xsh = jnp.concatenate([xw[:, :, kx:kx + 224, :]
                           for kx in range(3)], axis=3)
    xsh = jnp.pad(xsh, ((0, 0), (1, 15), (0, 0), (0, 0)))
    w1 = jnp.transpose(conv1_w, (2, 3, 1, 0)).reshape(27, DM)
    wp = jnp.pad(jnp.transpose(conv2_w, (1, 2, 3, 0)),
                 ((0, 0), (0, 0), (0, 0), (0, 1)))
    groups = {(0, 0): (0,), (0, 1): (1, 2), (1, 0): (0, 1), (1, 1): (2,)}
    w2 = jnp.stack([
        jnp.stack([
            jnp.stack([
                jnp.stack([
                    sum(wp[:, ky, kx] for ky in groups[py, ia]
                        for kx in groups[px, ja])
                    for ja in range(2)])
                for ia in range(2)])
            for px in range(2)])
        for py in range(2)])
    cbt = codebook.T
    g1 = bn1_g.reshape(1, DM)
    b1 = bn1_b.reshape(1, DM)
    g2 = jnp.pad(bn2_g, (0, 1)).reshape(1, 4)
    b2 = jnp.pad(bn2_b, (0, 1)).reshape(1, 4)
    grid = (56,)
    params = pltpu.CompilerParams(dimension_semantics=("arbitrary",))

    xablock = pl.BlockSpec((1, 16, 224, 9), lambda i: (i // 14, i % 14, 0, 0))
    xbblock = pl.BlockSpec((1, 16, 224, 9),
                           lambda i: (i // 14, i % 14 + 1, 0, 0))
    s1, s2 = pl.pallas_call(
        _stats1_kernel,
        grid=grid,
        in_specs=[xablock, xbblock, _full(w1.shape)],
        out_specs=[_full((1, DM)), _full((1, DM))],
        out_shape=[jax.ShapeDtypeStruct((1, DM), f32)] * 2,
        compiler_params=params,
    )(xsh, xsh, w1)

    idx, quant, loss = pl.pallas_call(
        _encode_kernel,
        grid=grid,
        in_specs=[xablock, xbblock, _full(w1.shape), _full((1, DM)),
                  _full((1, DM)), _full((1, DM)), _full((1, DM)),
                  _full(cbt.shape), _full(codebook.shape)],
        out_specs=[
            pl.BlockSpec((1, 1, 896), lambda i: (i, 0, 0)),
            pl.BlockSpec((1, 896, DM), lambda i: (i, 0, 0)),
            _full((1, 1)),
        ],
        out_shape=[
            jax.ShapeDtypeStruct((56, 1, 896), jnp.int32),
            jax.ShapeDtypeStruct((56, 896, DM), f32),
            jax.ShapeDtypeStruct((1, 1), f32),
        ],
        compiler_params=params,
    )(xsh, xsh, w1, g1, b1, s1, s2, cbt, codebook)

    indices = idx.reshape(4, 112, 112)
    commit_loss = loss[0, 0] / (NTOK * DM)
    qrows = quant.reshape(56, 8, 112, DM)

    qblk = pl.BlockSpec((1, 8, 112, DM), lambda i: (i, 0, 0, 0))
    qprev = pl.BlockSpec((1, 8, 112, DM),
                         lambda i: (jnp.maximum(i - 1, 0), 0, 0, 0))
    qnext = pl.BlockSpec((1, 8, 112, DM),
                         lambda i: (jnp.minimum(i + 1, 55), 0, 0, 0))
    pblk = pl.BlockSpec((1, 8, 112, 4), lambda i: (i // 14, i % 14, 0, 0))
    pshape = jax.ShapeDtypeStruct((4, 112, 112, 4), f32)

    r00, r01, r10, r11, t1, t2 = pl.pallas_call(
        _decode_kernel,
        grid=grid,
        in_specs=[qblk, qprev, qnext, _full(w2.shape)],
        out_specs=[pblk, pblk, pblk, pblk, _full((1, 4)), _full((1, 4))],
        out_shape=[pshape, pshape, pshape, pshape,
                   jax.ShapeDtypeStruct((1, 4), f32),
                   jax.ShapeDtypeStruct((1, 4), f32)],
        compiler_params=params,
    )(qrows, qrows, qrows, w2)

    f00, f01, f10, f11 = pl.pallas_call(
        _finish_kernel,
        grid=grid,
        in_specs=[pblk, pblk, pblk, pblk,
                  _full((1, 4)), _full((1, 4)), _full((1, 4)), _full((1, 4))],
        out_specs=[pblk, pblk, pblk, pblk],
        out_shape=[pshape, pshape, pshape, pshape],
        compiler_params=params,
    )(r00, r01, r10, r11, g2, b2, t1, t2)

    pp = jnp.stack([f00, f01, f10, f11]).reshape(2, 2, 4, 112, 112, 4)[..., :3]
    out = jnp.transpose(pp, (2, 5, 3, 0, 4, 1)).reshape(4, 3, 224, 224)
    return out, indices, commit_loss


# speedup vs baseline: 1.2887x; 1.2887x over previous
import jax
import jax.numpy as jnp
from jax.experimental import pallas as pl
from jax.experimental.pallas import tpu as pltpu

DM = 64
CB = 1024
BN_EPS = 1e-5
NPIX = 4 * 224 * 224
NTOK = 4 * 112 * 112


def _conv1_h(xa_ref, xb_ref, w1_ref):
    win = jnp.concatenate([xa_ref[0], xb_ref[0, :2]], axis=0)
    x27 = jnp.concatenate([win[ky:ky + 16] for ky in range(3)],
                          axis=2).reshape(16 * 224, 27)
    return jnp.dot(x27, w1_ref[...], preferred_element_type=jnp.float32)


def _stats1_kernel(xa_ref, xb_ref, w1_ref, s1_ref, s2_ref):
    i = pl.program_id(0)

    @pl.when(i == 0)
    def _():
        s1_ref[...] = jnp.zeros_like(s1_ref)
        s2_ref[...] = jnp.zeros_like(s2_ref)

    h = _conv1_h(xa_ref, xb_ref, w1_ref)
    s1_ref[...] += jnp.sum(h, axis=0, keepdims=True)
    s2_ref[...] += jnp.sum(h * h, axis=0, keepdims=True)


def _encode_kernel(xa_ref, xb_ref, w1_ref, g1_ref, b1_ref, s1_ref, s2_ref,
                   cbt_ref, cb_ref, idx_ref, quant_ref, loss_ref):
    i = pl.program_id(0)
    mean = s1_ref[...] / NPIX
    var = s2_ref[...] / NPIX - mean * mean
    scale = g1_ref[...] * jax.lax.rsqrt(var + BN_EPS)
    shift = b1_ref[...] - mean * scale

    h = _conv1_h(xa_ref, xb_ref, w1_ref)
    y = jnp.maximum(h * scale + shift, 0.0)
    y3 = y.reshape(16, 224, DM)
    p = jnp.max(y3.reshape(16, 112, 2, DM), axis=2)
    p = jnp.max(p.reshape(8, 2, 112, DM), axis=1)
    tok = p.reshape(896, DM)

    tsq = jnp.sum(tok * tok, axis=1, keepdims=True)
    csq = jnp.sum(cbt_ref[...] * cbt_ref[...], axis=0, keepdims=True)
    dist = tsq - jnp.dot(tok, cbt_ref[...] * 2.0,
                         preferred_element_type=jnp.float32) + csq
    mind = jnp.min(dist, axis=1, keepdims=True)
    lanes = jax.lax.broadcasted_iota(jnp.int32, dist.shape, 1)
    idx = jnp.min(jnp.where(dist == mind, lanes, jnp.int32(1 << 30)), axis=1)
    idx_ref[...] = idx.reshape(1, 1, 896)

    onehot = (lanes == idx[:, None]).astype(jnp.float32)
    quant_ref[...] = jnp.dot(onehot, cb_ref[...],
                             preferred_element_type=jnp.float32
                             ).reshape(1, 896, DM)

    @pl.when(i == 0)
    def _():
        loss_ref[...] = jnp.zeros_like(loss_ref)

    loss_ref[...] += jnp.sum(mind)


def _decode_kernel(q_ref, qprev_ref, qnext_ref, w2_ref,
                   p00_ref, p01_ref, p10_ref, p11_ref, s1_ref, s2_ref):
    i = pl.program_id(0)
    rc = i % 14
    cur = q_ref[0]
    top = jnp.where(rc > 0, qprev_ref[0, 7], 0.0)
    bot = jnp.where(rc < 13, qnext_ref[0, 0], 0.0)
    qwin = jnp.concatenate([top[None], cur, bot[None]], axis=0)
    zc = jnp.zeros((10, 1, DM), jnp.float32)
    qs = jnp.concatenate([zc, qwin, zc], axis=1)

    @pl.when(i == 0)
    def _():
        s1_ref[...] = jnp.zeros_like(s1_ref)
        s2_ref[...] = jnp.zeros_like(s2_ref)

    outs = [p00_ref, p01_ref, p10_ref, p11_ref]
    for py in range(2):
        for px in range(2):
            acc = jnp.zeros((8 * 112, 4), jnp.float32)
            for ia in range(2):
                for ja in range(2):
                    sl = jax.lax.slice(
                        qs, (py + ia, px + ja, 0),
                        (py + ia + 8, px + ja + 112, DM)).reshape(8 * 112, DM)
                    acc = acc + jnp.dot(sl, w2_ref[py, px, ia, ja],
                                        preferred_element_type=jnp.float32)
            outs[2 * py + px][...] = acc.reshape(1, 8, 112, 4)
            s1_ref[...] += jnp.sum(acc, axis=0, keepdims=True)
            s2_ref[...] += jnp.sum(acc * acc, axis=0, keepdims=True)


def _finish_kernel(r00_ref, r01_ref, r10_ref, r11_ref, g2_ref, b2_ref,
                   s1_ref, s2_ref, f00_ref, f01_ref, f10_ref, f11_ref):
    mean = s1_ref[...] / NPIX
    var = s2_ref[...] / NPIX - mean * mean
    scale = g2_ref[...] * jax.lax.rsqrt(var + BN_EPS)
    shift = b2_ref[...] - mean * scale
    f00_ref[...] = jnp.tanh(r00_ref[...] * scale + shift)
    f01_ref[...] = jnp.tanh(r01_ref[...] * scale + shift)
    f10_ref[...] = jnp.tanh(r10_ref[...] * scale + shift)
    f11_ref[...] = jnp.tanh(r11_ref[...] * scale + shift)


def _full(shape):
    n = len(shape)
    return pl.BlockSpec(shape, lambda i: (0,) * n)


def kernel(x, conv1_w, conv1_b, bn1_g, bn1_b, codebook,
           conv2_w, conv2_b, bn2_g, bn2_b):
    f32 = jnp.float32
    xt = jnp.transpose(x, (0, 2, 3, 1))
    xw = jnp.pad(xt, ((0, 0), (0, 0), (1, 1), (0, 0)))
    xsh = jnp.concatenate([xw[:, :, kx:kx + 224, :]
                           for kx in range(3)], axis=3)
    xsh = jnp.pad(xsh, ((0, 0), (1, 15), (0, 0), (0, 0)))
    w1 = jnp.transpose(conv1_w, (2, 3, 1, 0)).reshape(27, DM)
    wp = jnp.pad(jnp.transpose(conv2_w, (1, 2, 3, 0)),
                 ((0, 0), (0, 0), (0, 0), (0, 1)))
    groups = {(0, 0): (0,), (0, 1): (1, 2), (1, 0): (0, 1), (1, 1): (2,)}
    w2 = jnp.stack([
        jnp.stack([
            jnp.stack([
                jnp.stack([
                    sum(wp[:, ky, kx] for ky in groups[py, ia]
                        for kx in groups[px, ja])
                    for ja in range(2)])
                for ia in range(2)])
            for px in range(2)])
        for py in range(2)])
    cbt = codebook.T
    g1 = bn1_g.reshape(1, DM)
    b1 = bn1_b.reshape(1, DM)
    g2 = jnp.pad(bn2_g, (0, 1)).reshape(1, 4)
    b2 = jnp.pad(bn2_b, (0, 1)).reshape(1, 4)
    grid = (56,)
    params = pltpu.CompilerParams(dimension_semantics=("arbitrary",))

    xablock = pl.BlockSpec((1, 16, 224, 9), lambda i: (i // 14, i % 14, 0, 0))
    xbblock = pl.BlockSpec((1, 16, 224, 9),
                           lambda i: (i // 14, i % 14 + 1, 0, 0))
    s1, s2 = pl.pallas_call(
        _stats1_kernel,
        grid=grid,
        in_specs=[xablock, xbblock, _full(w1.shape)],
        out_specs=[_full((1, DM)), _full((1, DM))],
        out_shape=[jax.ShapeDtypeStruct((1, DM), f32)] * 2,
        compiler_params=params,
    )(xsh, xsh, w1)

    idx, quant, loss = pl.pallas_call(
        _encode_kernel,
        grid=grid,
        in_specs=[xablock, xbblock, _full(w1.shape), _full((1, DM)),
                  _full((1, DM)), _full((1, DM)), _full((1, DM)),
                  _full(cbt.shape), _full(codebook.shape)],
        out_specs=[
            pl.BlockSpec((1, 1, 896), lambda i: (i, 0, 0)),
            pl.BlockSpec((1, 896, DM), lambda i: (i, 0, 0)),
            _full((1, 1)),
        ],
        out_shape=[
            jax.ShapeDtypeStruct((56, 1, 896), jnp.int32),
            jax.ShapeDtypeStruct((56, 896, DM), f32),
            jax.ShapeDtypeStruct((1, 1), f32),
        ],
        compiler_params=params,
    )(xsh, xsh, w1, g1, b1, s1, s2, cbt, codebook)

    indices = idx.reshape(4, 112, 112)
    commit_loss = loss[0, 0] / (NTOK * DM)
    qrows = quant.reshape(56, 8, 112, DM)

    qblk = pl.BlockSpec((1, 8, 112, DM), lambda i: (i, 0, 0, 0))
    qprev = pl.BlockSpec((1, 8, 112, DM),
                         lambda i: (jnp.maximum(i - 1, 0), 0, 0, 0))
    qnext = pl.BlockSpec((1, 8, 112, DM),
                         lambda i: (jnp.minimum(i + 1, 55), 0, 0, 0))
    pblk = pl.BlockSpec((1, 8, 112, 4), lambda i: (i // 14, i % 14, 0, 0))
    pshape = jax.ShapeDtypeStruct((4, 112, 112, 4), f32)

    r00, r01, r10, r11, t1, t2 = pl.pallas_call(
        _decode_kernel,
        grid=grid,
        in_specs=[qblk, qprev, qnext, _full(w2.shape)],
        out_specs=[pblk, pblk, pblk, pblk, _full((1, 4)), _full((1, 4))],
        out_shape=[pshape, pshape, pshape, pshape,
                   jax.ShapeDtypeStruct((1, 4), f32),
                   jax.ShapeDtypeStruct((1, 4), f32)],
        compiler_params=params,
    )(qrows, qrows, qrows, w2)

    f00, f01, f10, f11 = pl.pallas_call(
        _finish_kernel,
        grid=grid,
        in_specs=[pblk, pblk, pblk, pblk,
                  _full((1, 4)), _full((1, 4)), _full((1, 4)), _full((1, 4))],
        out_specs=[pblk, pblk, pblk, pblk],
        out_shape=[pshape, pshape, pshape, pshape],
        compiler_params=params,
    )(r00, r01, r10, r11, g2, b2, t1, t2)

    pp = jnp.stack([f00, f01, f10, f11]).reshape(2, 2, 4, 112, 112, 4)[..., :3]
    out = jnp.transpose(pp, (2, 5, 3, 0, 4, 1)).reshape(4, 3, 224, 224)
    return out, indices, commit_loss
